# baseline (speedup 1.0000x reference)
import functools

import jax
import jax.numpy as jnp
from jax import lax
from jax.experimental import pallas as pl
from jax.experimental.pallas import tpu as pltpu
from jax.experimental.pallas import tpu_sc as plsc

N_NODES = 10000
N_EDGES = 320000
F = 128

NC = 2
NS = 16
NW = NC * NS

E_PER_TILE = N_EDGES // NW
VREGS = E_PER_TILE // 16
CHUNK = 128
C_FULL = E_PER_TILE // CHUNK

C_SIZE = N_NODES * F
SCRAP = C_SIZE
ACC = 1281280
ACC_PER_TILE = ACC // NS
ZCHUNK = ACC_PER_TILE // 10
DEG_PAD = 10016


def _sc_body(edge_ref, feat_ref, acc_out, deg_out,
             feat_v, ebuf_v, idx2d, ones_v, zbuf_v, deg_v, accsp, sem):
    cid = lax.axis_index("c")
    sid = lax.axis_index("s")
    wid = cid * NS + sid

    def zfill(i, carry):
        zbuf_v[pl.ds(i * 16, 16)] = jnp.zeros((16,), jnp.float32)
        return carry
    lax.fori_loop(0, ZCHUNK // 16, zfill, 0)
    def dzero(i, carry):
        deg_v[pl.ds(i * 16, 16)] = jnp.zeros((16,), jnp.float32)
        return carry
    lax.fori_loop(0, DEG_PAD // 16, dzero, 0)
    ones16 = jnp.ones((16,), jnp.float32)
    for k in range(CHUNK // 16):
        ones_v[pl.ds(k * 16, 16)] = ones16

    def zero_acc(k, carry):
        pltpu.sync_copy(zbuf_v,
                        accsp.at[pl.ds(sid * ACC_PER_TILE + k * ZCHUNK,
                                       ZCHUNK)])
        return carry
    lax.fori_loop(0, ACC_PER_TILE // ZCHUNK, zero_acc, 0)

    pltpu.sync_copy(feat_ref, feat_v)
    pltpu.sync_copy(edge_ref.at[pl.ds(wid * (2 * E_PER_TILE),
                                      2 * E_PER_TILE)], ebuf_v)

    def dhist(j, carry):
        s16 = ebuf_v[pl.ds(j * 16, 16)]
        cnt, last = plsc.scan_count(s16)
        plsc.addupdate_scatter(deg_v, [s16], cnt.astype(jnp.float32),
                               mask=last)
        return carry
    lax.fori_loop(0, VREGS, dhist, 0)

    def c_key(i):
        s16 = ebuf_v[pl.ds(i * 16, 16)]
        d16 = ebuf_v[pl.ds(E_PER_TILE + i * 16, 16)]
        f16 = plsc.load_gather(feat_v, [s16])
        return d16 * F + f16

    plsc.subcore_barrier()

    def fire(slot):
        pltpu.async_copy(ones_v, accsp.at[idx2d.at[slot]], sem, add=True)

    def drain():
        pltpu.make_async_copy(ones_v, accsp.at[idx2d.at[0]], sem).wait()

    scrap16 = jnp.full((16,), SCRAP, jnp.int32)

    for s in range(8):
        for v in range(8):
            idx2d[s, pl.ds(v * 16, 16)] = c_key(s * 8 + v)
        fire(s)

    def step(r, carry):
        drain()
        slot = r % 8
        for v in range(8):
            idx2d[slot, pl.ds(v * 16, 16)] = c_key(r * 8 + v)
        fire(slot)
        return carry
    lax.fori_loop(8, C_FULL, step, 0)

    drain()
    tslot = C_FULL % 8
    for v in range(8):
        idx2d[tslot, pl.ds(v * 16, 16)] = (c_key(C_FULL * 8 + v) if v < 1
                                           else scrap16)
    fire(tslot)
    for _ in range(8):
        drain()

    plsc.subcore_barrier()

    pltpu.sync_copy(deg_v, deg_out.at[pl.ds(wid * DEG_PAD, DEG_PAD)])

    def wout(k, carry):
        off = sid * ACC_PER_TILE + k * ZCHUNK
        pltpu.sync_copy(accsp.at[pl.ds(off, ZCHUNK)], zbuf_v)
        pltpu.sync_copy(zbuf_v, acc_out.at[pl.ds(cid * ACC + off, ZCHUNK)])
        return carry
    lax.fori_loop(0, ACC_PER_TILE // ZCHUNK, wout, 0)


@functools.cache
def _sc_histograms():
  return pl.kernel(
    _sc_body,
    out_type=(jax.ShapeDtypeStruct((NC * ACC,), jnp.float32),
              jax.ShapeDtypeStruct((NW * DEG_PAD,), jnp.float32)),
    mesh=plsc.VectorSubcoreMesh(core_axis_name="c", subcore_axis_name="s"),
    scratch_types=[
        pltpu.VMEM((N_NODES,), jnp.int32),
        pltpu.VMEM((2 * E_PER_TILE,), jnp.int32),
        pltpu.VMEM((8, CHUNK), jnp.int32),
        pltpu.VMEM((CHUNK,), jnp.float32),
        pltpu.VMEM((ZCHUNK,), jnp.float32),
        pltpu.VMEM((DEG_PAD,), jnp.float32),
        pltpu.VMEM_SHARED((ACC,), jnp.float32),
        pltpu.SemaphoreType.DMA,
    ],
    compiler_params=pltpu.CompilerParams(needs_layout_passes=False),
  )


def _tc_body(cp_ref, degp_ref, feat_ref, emb_ref, w1_ref, b1_ref,
             w2_ref, b2_ref, out_ref):
    hi = jax.lax.Precision.HIGHEST
    emb1 = jnp.dot(emb_ref[...], w1_ref[...], precision=hi)
    col = lax.broadcasted_iota(jnp.int32, (N_NODES, F), 1)
    oh = (feat_ref[...] == col).astype(jnp.float32)
    d = cp_ref[0] + cp_ref[1] + oh
    z = jnp.dot(d, emb1, precision=hi) + b1_ref[...]
    h1 = jnp.maximum(z, 0.0)
    deg = jnp.sum(degp_ref[...], axis=0, keepdims=True)
    wrow = (deg + 1.0) * (1.0 / N_NODES)
    s = jnp.dot(wrow, h1, precision=hi)
    out_ref[...] = jnp.dot(s, w2_ref[...], precision=hi) + b2_ref[...]


_tc_dense = pl.pallas_call(
    _tc_body,
    out_shape=jax.ShapeDtypeStruct((1, F), jnp.float32),
)


@jax.jit
def kernel(in_feat, edge_index, emb, W1, b1, W2, b2):
    feat = in_feat.astype(jnp.int32)
    edge_il = (edge_index.astype(jnp.int32)
               .reshape(2, NW, E_PER_TILE)
               .transpose(1, 0, 2)
               .reshape(NW * 2 * E_PER_TILE))
    acc, deg = _sc_histograms()(edge_il, feat)
    cp = acc.reshape(NC, ACC)[:, :C_SIZE].reshape(NC, N_NODES, F)
    degp = deg.reshape(NW, DEG_PAD)[:, :N_NODES]
    out = _tc_dense(cp, degp, feat.reshape(N_NODES, 1), emb, W1,
                    b1.reshape(1, F), W2, b2.reshape(1, F))
    return out.reshape(F)


# speedup vs baseline: 1.4778x; 1.4778x over previous
import functools

import jax
import jax.numpy as jnp
from jax import lax
from jax.experimental import pallas as pl
from jax.experimental.pallas import tpu as pltpu
from jax.experimental.pallas import tpu_sc as plsc

N_NODES = 10000
N_EDGES = 320000
F = 128

NC = 2
NS = 16
NW = NC * NS

E_PER_TILE = N_EDGES // NW
VREGS = E_PER_TILE // 16
CHUNK = 128
C_FULL = E_PER_TILE // CHUNK

C_SIZE = N_NODES * F
ACC = C_SIZE
ACC_PER_TILE = ACC // NS
ZCHUNK = ACC_PER_TILE // 10
DEG_PAD = 10016


def _sc_body(edge_ref, feat_ref, acc_out, deg_out,
             feat_v, ebuf_v, idx2d, ones_v, tvals_v, zbuf_v, deg_v,
             accsp, sem):
    cid = lax.axis_index("c")
    sid = lax.axis_index("s")
    wid = cid * NS + sid

    def zfill(i, carry):
        zbuf_v[pl.ds(i * 16, 16)] = jnp.zeros((16,), jnp.float32)
        return carry
    lax.fori_loop(0, ZCHUNK // 16, zfill, 0)
    def dzero(i, carry):
        deg_v[pl.ds(i * 16, 16)] = jnp.zeros((16,), jnp.float32)
        return carry
    lax.fori_loop(0, DEG_PAD // 16, dzero, 0)
    ones16 = jnp.ones((16,), jnp.float32)
    zeros16 = jnp.zeros((16,), jnp.float32)
    for k in range(CHUNK // 16):
        ones_v[pl.ds(k * 16, 16)] = ones16
        tvals_v[pl.ds(k * 16, 16)] = ones16 if k < 1 else zeros16

    def zero_acc(k, carry):
        pltpu.sync_copy(zbuf_v,
                        accsp.at[pl.ds(sid * ACC_PER_TILE + k * ZCHUNK,
                                       ZCHUNK)])
        return carry
    lax.fori_loop(0, ACC_PER_TILE // ZCHUNK, zero_acc, 0)

    pltpu.sync_copy(feat_ref, feat_v)
    pltpu.sync_copy(edge_ref.at[pl.ds(wid * E_PER_TILE, E_PER_TILE)],
                    ebuf_v.at[pl.ds(0, E_PER_TILE)])
    pltpu.sync_copy(edge_ref.at[pl.ds(N_EDGES + wid * E_PER_TILE,
                                      E_PER_TILE)],
                    ebuf_v.at[pl.ds(E_PER_TILE, E_PER_TILE)])

    def dhist(j, carry):
        s16 = ebuf_v[pl.ds(j * 16, 16)]
        cnt, last = plsc.scan_count(s16)
        plsc.addupdate_scatter(deg_v, [s16], cnt.astype(jnp.float32),
                               mask=last)
        return carry
    lax.fori_loop(0, VREGS, dhist, 0)

    def c_key(i):
        s16 = ebuf_v[pl.ds(i * 16, 16)]
        d16 = ebuf_v[pl.ds(E_PER_TILE + i * 16, 16)]
        f16 = plsc.load_gather(feat_v, [s16])
        return d16 * F + f16

    plsc.subcore_barrier()

    def fire(slot, vals=None):
        pltpu.async_copy(ones_v if vals is None else vals,
                         accsp.at[idx2d.at[slot]], sem, add=True)

    def drain():
        pltpu.make_async_copy(ones_v, accsp.at[idx2d.at[0]], sem).wait()

    pad16 = jnp.zeros((16,), jnp.int32)

    for s in range(8):
        for v in range(8):
            idx2d[s, pl.ds(v * 16, 16)] = c_key(s * 8 + v)
        fire(s)

    def step(r, carry):
        drain()
        slot = r % 8
        for v in range(8):
            idx2d[slot, pl.ds(v * 16, 16)] = c_key(r * 8 + v)
        fire(slot)
        return carry
    lax.fori_loop(8, C_FULL, step, 0)

    drain()
    tslot = C_FULL % 8
    for v in range(8):
        idx2d[tslot, pl.ds(v * 16, 16)] = (c_key(C_FULL * 8 + v) if v < 1
                                           else pad16)
    fire(tslot, tvals_v)
    for _ in range(8):
        drain()

    plsc.subcore_barrier()

    pltpu.sync_copy(deg_v.at[pl.ds(0, N_NODES)],
                    deg_out.at[pl.ds(wid * N_NODES, N_NODES)])

    def wout(k, carry):
        off = sid * ACC_PER_TILE + k * ZCHUNK
        pltpu.sync_copy(accsp.at[pl.ds(off, ZCHUNK)], zbuf_v)
        pltpu.sync_copy(zbuf_v, acc_out.at[pl.ds(cid * ACC + off, ZCHUNK)])
        return carry
    lax.fori_loop(0, ACC_PER_TILE // ZCHUNK, wout, 0)


@functools.cache
def _sc_histograms():
  return pl.kernel(
    _sc_body,
    out_type=(jax.ShapeDtypeStruct((NC * ACC,), jnp.float32),
              jax.ShapeDtypeStruct((NW * N_NODES,), jnp.float32)),
    mesh=plsc.VectorSubcoreMesh(core_axis_name="c", subcore_axis_name="s"),
    scratch_types=[
        pltpu.VMEM((N_NODES,), jnp.int32),
        pltpu.VMEM((2 * E_PER_TILE,), jnp.int32),
        pltpu.VMEM((8, CHUNK), jnp.int32),
        pltpu.VMEM((CHUNK,), jnp.float32),
        pltpu.VMEM((CHUNK,), jnp.float32),
        pltpu.VMEM((ZCHUNK,), jnp.float32),
        pltpu.VMEM((DEG_PAD,), jnp.float32),
        pltpu.VMEM_SHARED((ACC,), jnp.float32),
        pltpu.SemaphoreType.DMA,
    ],
    compiler_params=pltpu.CompilerParams(needs_layout_passes=False),
  )


def _tc_body(cp_ref, degp_ref, feat_ref, emb_ref, w1_ref, b1_ref,
             w2_ref, b2_ref, out_ref):
    hi = jax.lax.Precision.HIGHEST
    emb1 = jnp.dot(emb_ref[...], w1_ref[...], precision=hi)
    col = lax.broadcasted_iota(jnp.int32, (N_NODES, F), 1)
    oh = (feat_ref[...] == col).astype(jnp.float32)
    d = cp_ref[0] + cp_ref[1] + oh
    z = jnp.dot(d, emb1, precision=hi) + b1_ref[...]
    h1 = jnp.maximum(z, 0.0)
    deg = jnp.sum(degp_ref[...], axis=0, keepdims=True)
    wrow = (deg + 1.0) * (1.0 / N_NODES)
    s = jnp.dot(wrow, h1, precision=hi)
    out_ref[...] = jnp.dot(s, w2_ref[...], precision=hi) + b2_ref[...]


_tc_dense = pl.pallas_call(
    _tc_body,
    out_shape=jax.ShapeDtypeStruct((1, F), jnp.float32),
)


@jax.jit
def kernel(in_feat, edge_index, emb, W1, b1, W2, b2):
    feat = in_feat.astype(jnp.int32)
    edge_flat = edge_index.astype(jnp.int32).reshape(2 * N_EDGES)
    acc, deg = _sc_histograms()(edge_flat, feat)
    cp = acc.reshape(NC, N_NODES, F)
    degp = deg.reshape(NW, N_NODES)
    out = _tc_dense(cp, degp, feat.reshape(N_NODES, 1), emb, W1,
                    b1.reshape(1, F), W2, b2.reshape(1, F))
    return out.reshape(F)


# speedup vs baseline: 1.5832x; 1.0713x over previous
import functools

import jax
import jax.numpy as jnp
from jax import lax
from jax.experimental import pallas as pl
from jax.experimental.pallas import tpu as pltpu
from jax.experimental.pallas import tpu_sc as plsc

N_NODES = 10000
N_EDGES = 320000
F = 128

NC = 2
NS = 16
NW = NC * NS

E_PER_TILE = N_EDGES // NW
VREGS = E_PER_TILE // 16
CHUNK = 128
C_FULL = E_PER_TILE // CHUNK

C_SIZE = N_NODES * F
ACC = C_SIZE
ACC_PER_TILE = ACC // NS
ZCHUNK = ACC_PER_TILE // 10
DEG_PAD = 10016


def _sc_body(edge_ref, feat_ref, acc_out, deg_out,
             feat_v, ebuf_v, idx2d, ones_v, tvals_v, zbuf_v, deg_v,
             accsp, sem, semz, semw):
    cid = lax.axis_index("c")
    sid = lax.axis_index("s")
    wid = cid * NS + sid

    def zfill(i, carry):
        zbuf_v[pl.ds(i * 16, 16)] = jnp.zeros((16,), jnp.float32)
        return carry
    lax.fori_loop(0, ZCHUNK // 16, zfill, 0)
    def dzero(i, carry):
        deg_v[pl.ds(i * 16, 16)] = jnp.zeros((16,), jnp.float32)
        return carry
    lax.fori_loop(0, DEG_PAD // 16, dzero, 0)
    ones16 = jnp.ones((16,), jnp.float32)
    zeros16 = jnp.zeros((16,), jnp.float32)
    for k in range(CHUNK // 16):
        ones_v[pl.ds(k * 16, 16)] = ones16
        tvals_v[pl.ds(k * 16, 16)] = ones16 if k < 1 else zeros16

    nz = ACC_PER_TILE // ZCHUNK
    def zero_acc(k, carry):
        pltpu.async_copy(zbuf_v,
                         accsp.at[pl.ds(sid * ACC_PER_TILE + k * ZCHUNK,
                                        ZCHUNK)], semz)
        return carry
    lax.fori_loop(0, nz, zero_acc, 0)

    pltpu.sync_copy(feat_ref, feat_v)
    pltpu.sync_copy(edge_ref.at[pl.ds(wid * E_PER_TILE, E_PER_TILE)],
                    ebuf_v.at[pl.ds(0, E_PER_TILE)])
    pltpu.sync_copy(edge_ref.at[pl.ds(N_EDGES + wid * E_PER_TILE,
                                      E_PER_TILE)],
                    ebuf_v.at[pl.ds(E_PER_TILE, E_PER_TILE)])

    def dhist(j, carry):
        s16 = ebuf_v[pl.ds(j * 16, 16)]
        cnt, last = plsc.scan_count(s16)
        plsc.addupdate_scatter(deg_v, [s16], cnt.astype(jnp.float32),
                               mask=last)
        return carry
    lax.fori_loop(0, VREGS, dhist, 0)

    def c_key(i):
        s16 = ebuf_v[pl.ds(i * 16, 16)]
        d16 = ebuf_v[pl.ds(E_PER_TILE + i * 16, 16)]
        f16 = plsc.load_gather(feat_v, [s16])
        return d16 * F + f16

    def zdrain(k, carry):
        pltpu.make_async_copy(
            zbuf_v, accsp.at[pl.ds(sid * ACC_PER_TILE, ZCHUNK)], semz).wait()
        return carry
    lax.fori_loop(0, nz, zdrain, 0)
    plsc.subcore_barrier()

    def fire(slot, vals=None):
        pltpu.async_copy(ones_v if vals is None else vals,
                         accsp.at[idx2d.at[slot]], sem, add=True)

    def drain():
        pltpu.make_async_copy(ones_v, accsp.at[idx2d.at[0]], sem).wait()

    pad16 = jnp.zeros((16,), jnp.int32)

    for s in range(8):
        for v in range(8):
            idx2d[s, pl.ds(v * 16, 16)] = c_key(s * 8 + v)
        fire(s)

    def step(r, carry):
        drain()
        slot = r % 8
        for v in range(8):
            idx2d[slot, pl.ds(v * 16, 16)] = c_key(r * 8 + v)
        fire(slot)
        return carry
    lax.fori_loop(8, C_FULL, step, 0)

    drain()
    tslot = C_FULL % 8
    for v in range(8):
        idx2d[tslot, pl.ds(v * 16, 16)] = (c_key(C_FULL * 8 + v) if v < 1
                                           else pad16)
    fire(tslot, tvals_v)
    for _ in range(8):
        drain()

    plsc.subcore_barrier()

    pltpu.sync_copy(deg_v.at[pl.ds(0, N_NODES)],
                    deg_out.at[pl.ds(wid * N_NODES, N_NODES)])

    zh = ZCHUNK // 2
    nw = ACC_PER_TILE // zh

    def rd_slice(k):
        return (accsp.at[pl.ds(sid * ACC_PER_TILE + k * zh, zh)],
                zbuf_v.at[pl.ds((k % 2) * zh, zh)])

    def wr_slice(k):
        return (zbuf_v.at[pl.ds((k % 2) * zh, zh)],
                acc_out.at[pl.ds(cid * ACC + sid * ACC_PER_TILE + k * zh,
                                 zh)])

    pltpu.async_copy(*rd_slice(0), semz)
    for k in range(nw):
        pltpu.make_async_copy(*rd_slice(k), semz).wait()
        if k >= 1:
            pltpu.make_async_copy(*wr_slice(k - 1), semw).wait()
        if k + 1 < nw:
            pltpu.async_copy(*rd_slice(k + 1), semz)
        pltpu.async_copy(*wr_slice(k), semw)
    pltpu.make_async_copy(*wr_slice(nw - 1), semw).wait()


@functools.cache
def _sc_histograms():
  return pl.kernel(
    _sc_body,
    out_type=(jax.ShapeDtypeStruct((NC * ACC,), jnp.float32),
              jax.ShapeDtypeStruct((NW * N_NODES,), jnp.float32)),
    mesh=plsc.VectorSubcoreMesh(core_axis_name="c", subcore_axis_name="s"),
    scratch_types=[
        pltpu.VMEM((N_NODES,), jnp.int32),
        pltpu.VMEM((2 * E_PER_TILE,), jnp.int32),
        pltpu.VMEM((8, CHUNK), jnp.int32),
        pltpu.VMEM((CHUNK,), jnp.float32),
        pltpu.VMEM((CHUNK,), jnp.float32),
        pltpu.VMEM((ZCHUNK,), jnp.float32),
        pltpu.VMEM((DEG_PAD,), jnp.float32),
        pltpu.VMEM_SHARED((ACC,), jnp.float32),
        pltpu.SemaphoreType.DMA,
        pltpu.SemaphoreType.DMA,
        pltpu.SemaphoreType.DMA,
    ],
    compiler_params=pltpu.CompilerParams(needs_layout_passes=False),
  )


def _tc_body(cp_ref, degp_ref, feat_ref, emb_ref, w1_ref, b1_ref,
             w2_ref, b2_ref, out_ref):
    hi = jax.lax.Precision.HIGHEST
    emb1 = jnp.dot(emb_ref[...], w1_ref[...], precision=hi)
    col = lax.broadcasted_iota(jnp.int32, (N_NODES, F), 1)
    oh = (feat_ref[...] == col).astype(jnp.float32)
    d = cp_ref[0] + cp_ref[1] + oh
    z = jnp.dot(d, emb1, precision=hi) + b1_ref[...]
    h1 = jnp.maximum(z, 0.0)
    deg = jnp.sum(degp_ref[...], axis=0, keepdims=True)
    wrow = (deg + 1.0) * (1.0 / N_NODES)
    s = jnp.dot(wrow, h1, precision=hi)
    out_ref[...] = jnp.dot(s, w2_ref[...], precision=hi) + b2_ref[...]


_tc_dense = pl.pallas_call(
    _tc_body,
    out_shape=jax.ShapeDtypeStruct((1, F), jnp.float32),
)


@jax.jit
def kernel(in_feat, edge_index, emb, W1, b1, W2, b2):
    feat = in_feat.astype(jnp.int32)
    edge_flat = edge_index.astype(jnp.int32).reshape(2 * N_EDGES)
    acc, deg = _sc_histograms()(edge_flat, feat)
    cp = acc.reshape(NC, N_NODES, F)
    degp = deg.reshape(NW, N_NODES)
    out = _tc_dense(cp, degp, feat.reshape(N_NODES, 1), emb, W1,
                    b1.reshape(1, F), W2, b2.reshape(1, F))
    return out.reshape(F)
